# baseline (speedup 1.0000x reference)
import functools

import jax
import jax.numpy as jnp
from jax import lax
from jax.experimental import pallas as pl
from jax.experimental.pallas import tpu as pltpu
from jax.experimental.pallas import tpu_sc as plsc

B = 64
N = 32768
K = 2048
L = 16
NV = N // L
NW = 32
ROWS_PER_W = B // NW

W0, W1, W2 = 11, 10, 10
NB0, NB1, NB2 = 1 << W0, 1 << W1, 1 << W2
STRIDE = NB0
HIST_WORDS = STRIDE * L

TG = 1.25
CAP = 352
CAND_WORDS = L * CAP + NV
WC = 8
NBC = 1 << WC

INT_MAX = 2**31 - 1


def _row_topk(row_v, hist_v, tot_v, csum_v, cand_v, lane):
    lane_off = lane * STRIDE
    ones = jnp.ones((L,), jnp.int32)
    zeros = jnp.zeros((L,), jnp.int32)

    def hist_from_row(shift, width, prefix):
        nb = 1 << width
        base = 0 if prefix is None else prefix * nb

        @plsc.parallel_loop(0, NV, unroll=16)
        def _(i):
            v = row_v[pl.ds(i * L, L)]
            bits = lax.bitcast_convert_type(v, jnp.uint32)
            d = (lax.shift_right_logical(bits, jnp.uint32(shift))
                 - jnp.uint32(base)).astype(jnp.int32)
            m = d.astype(jnp.uint32) < jnp.uint32(nb)
            plsc.addupdate_scatter(hist_v, [lane_off + d], ones, mask=m)

    def hist_from_cand(shift, prefix, cursor, trip):
        base = 0 if prefix is None else prefix * NBC
        lane_cap = lane * CAP

        @plsc.parallel_loop(0, trip, unroll=4)
        def _(j):
            raw = plsc.load_gather(cand_v, [lane_cap + j])
            bits = lax.bitcast_convert_type(raw, jnp.uint32)
            d = (lax.shift_right_logical(bits, jnp.uint32(shift))
                 - jnp.uint32(base)).astype(jnp.int32)
            m = (d.astype(jnp.uint32) < jnp.uint32(NBC)) & (j < cursor)
            plsc.addupdate_scatter(hist_v, [lane_off + d], ones, mask=m)

    def reduce_level(width):
        nb = 1 << width

        @plsc.parallel_loop(0, nb // L, unroll=4)
        def _(c):
            acc = hist_v[pl.ds(c * L, L)]
            hist_v[pl.ds(c * L, L)] = zeros
            for l in range(1, L):
                acc = acc + hist_v[pl.ds(l * STRIDE + c * L, L)]
                hist_v[pl.ds(l * STRIDE + c * L, L)] = zeros
            tot_v[pl.ds(c * L, L)] = acc
            s = jnp.sum(acc)
            cvec = jnp.full((L,), c, jnp.int32)
            svec = jnp.full((L,), s, jnp.int32)
            plsc.store_scatter(csum_v, [cvec], svec, mask=lane == 0)

    def find_bucket(width, k_rem):
        nch = (1 << width) // L
        nchv = nch // L

        def body(i, carry):
            cnt, above = carry
            cv = nchv - 1 - i
            v = csum_v[pl.ds(cv * L, L)]
            suf = lax.rev(plsc.cumsum(lax.rev(v, (0,))), (0,)) + above
            cnt = cnt + jnp.sum(jnp.where(suf >= k_rem, 1, 0))
            above = above + jnp.sum(v)
            return cnt, above

        cnt, _ = lax.fori_loop(0, nchv, body, (jnp.int32(0), jnp.int32(0)))
        c0 = cnt - 1

        def body2(cv, acc):
            v = csum_v[pl.ds(cv * L, L)]
            g = cv * L + lane
            return acc + jnp.sum(jnp.where(g > c0, v, 0))

        above_c0 = lax.fori_loop(0, nchv, body2, jnp.int32(0))

        v = tot_v[pl.ds(c0 * L, L)]
        suf = lax.rev(plsc.cumsum(lax.rev(v, (0,))), (0,)) + above_c0
        m = suf >= k_rem
        pc = jnp.sum(jnp.where(m, 1, 0))
        b0 = c0 * L + pc - 1
        s_b0 = jnp.min(jnp.where(m, suf, INT_MAX))
        v_b0 = jnp.sum(jnp.where(lane == pc - 1, v, 0))
        return b0, k_rem - (s_b0 - v_b0)

    lane_cap = lane * CAP

    @plsc.parallel_loop(0, NV, unroll=8, carry=jnp.zeros((L,), jnp.int32))
    def compact(i, cursor):
        v = row_v[pl.ds(i * L, L)]
        m = v >= TG
        bits = lax.bitcast_convert_type(v, jnp.int32)
        plsc.store_scatter(cand_v, [lane_cap + cursor], bits, mask=m)
        return cursor + m.astype(jnp.int32)

    cursor = compact
    n_cand = jnp.sum(cursor)
    max_cand = jnp.max(cursor)
    k0 = jnp.int32(K)

    def cand_select(_):
        trip = jnp.minimum(max_cand, CAP)
        hist_from_cand(24, None, cursor, trip)
        reduce_level(WC)
        b0, k1 = find_bucket(WC, k0)
        hist_from_cand(16, b0, cursor, trip)
        reduce_level(WC)
        b1, k2 = find_bucket(WC, k1)
        p1 = b0 * NBC + b1
        hist_from_cand(8, p1, cursor, trip)
        reduce_level(WC)
        b2, k3 = find_bucket(WC, k2)
        p2 = p1 * NBC + b2
        hist_from_cand(0, p2, cursor, trip)
        reduce_level(WC)
        b3, _ = find_bucket(WC, k3)
        return p2 * NBC + b3

    def full_select(_):
        hist_from_row(W1 + W2, W0, None)
        b0, k1 = find_bucket(W0, k0)
        hist_from_row(W2, W1, b0)
        b1, k2 = find_bucket(W1, k1)
        p1 = b0 * NB1 + b1
        hist_from_row(0, W2, p1)
        b2, _ = find_bucket(W2, k2)
        return p1 * NB2 + b2

    ok = jnp.logical_and(n_cand >= k0, max_cand <= CAP)
    tbits = lax.cond(ok, cand_select, full_select, 0)
    t = lax.bitcast_convert_type(tbits, jnp.float32)

    @plsc.parallel_loop(0, NV, unroll=16)
    def _(i):
        v = row_v[pl.ds(i * L, L)]
        row_v[pl.ds(i * L, L)] = jnp.where(v >= t, v, 0.0)


@functools.partial(
    pl.kernel,
    out_type=jax.ShapeDtypeStruct((B, N), jnp.float32),
    mesh=plsc.VectorSubcoreMesh(core_axis_name="c", subcore_axis_name="s"),
    compiler_params=pltpu.CompilerParams(needs_layout_passes=False),
    scratch_types=[
        pltpu.VMEM((N,), jnp.float32),
        pltpu.VMEM((N,), jnp.float32),
        pltpu.VMEM((HIST_WORDS,), jnp.int32),
        pltpu.VMEM((NB0,), jnp.int32),
        pltpu.VMEM((NB0 // L,), jnp.int32),
        pltpu.VMEM((CAND_WORDS,), jnp.int32),
        pltpu.SemaphoreType.DMA,
        pltpu.SemaphoreType.DMA,
        pltpu.SemaphoreType.DMA,
    ],
)
def _topk_mask_sc(x_hbm, out_hbm, row_a, row_b, hist_v, tot_v, csum_v,
                  cand_v, sem_in0, sem_in1, sem_out):
    wid = lax.axis_index("s") * 2 + lax.axis_index("c")
    lane = lax.iota(jnp.int32, L)
    row0 = wid * ROWS_PER_W
    row1 = row0 + 1

    cp_in0 = pltpu.async_copy(x_hbm.at[row0], row_a, sem_in0)
    cp_in1 = pltpu.async_copy(x_hbm.at[row1], row_b, sem_in1)

    zeros = jnp.zeros((L,), jnp.int32)

    @plsc.parallel_loop(0, HIST_WORDS // L, unroll=8)
    def _(j):
        hist_v[pl.ds(j * L, L)] = zeros

    cp_in0.wait()
    _row_topk(row_a, hist_v, tot_v, csum_v, cand_v, lane)
    cp_out0 = pltpu.async_copy(row_a, out_hbm.at[row0], sem_out)
    cp_in1.wait()
    _row_topk(row_b, hist_v, tot_v, csum_v, cand_v, lane)
    pltpu.sync_copy(row_b, out_hbm.at[row1])
    cp_out0.wait()


def kernel(x):
    return _topk_mask_sc(x)


# speedup vs baseline: 1.2726x; 1.2726x over previous
import functools

import jax
import jax.numpy as jnp
from jax import lax
from jax.experimental import pallas as pl
from jax.experimental.pallas import tpu as pltpu
from jax.experimental.pallas import tpu_sc as plsc

B = 64
N = 32768
K = 2048
L = 16
NV = N // L
NW = 32
ROWS_PER_W = B // NW

W0, W1, W2 = 11, 10, 10
NB0, NB1, NB2 = 1 << W0, 1 << W1, 1 << W2
STRIDE = NB0 + 1
HIST_WORDS = STRIDE * L

TG = 1.25
CAP = 353
CAND_WORDS = L * CAP + NV
WC = 8
NBC = 1 << WC

INT_MAX = 2**31 - 1


def _row_topk(row_v, hist_v, tot_v, csum_v, cand_v, lane):
    lane_off = lane * STRIDE
    ones = jnp.ones((L,), jnp.int32)
    zeros = jnp.zeros((L,), jnp.int32)

    def hist_from_row(shift, width, prefix):
        nb = 1 << width
        base = 0 if prefix is None else prefix * nb

        @plsc.parallel_loop(0, NV, unroll=16)
        def _(i):
            v = row_v[pl.ds(i * L, L)]
            bits = lax.bitcast_convert_type(v, jnp.uint32)
            d = (lax.shift_right_logical(bits, jnp.uint32(shift))
                 - jnp.uint32(base)).astype(jnp.int32)
            m = d.astype(jnp.uint32) < jnp.uint32(nb)
            plsc.addupdate_scatter(hist_v, [lane_off + d], ones, mask=m)

    def hist_from_cand(shift, prefix, cursor, trip):
        base = 0 if prefix is None else prefix * NBC
        lane_cap = lane * CAP

        @plsc.parallel_loop(0, trip, unroll=4)
        def _(j):
            raw = plsc.load_gather(cand_v, [lane_cap + j])
            bits = lax.bitcast_convert_type(raw, jnp.uint32)
            d = (lax.shift_right_logical(bits, jnp.uint32(shift))
                 - jnp.uint32(base)).astype(jnp.int32)
            m = (d.astype(jnp.uint32) < jnp.uint32(NBC)) & (j < cursor)
            plsc.addupdate_scatter(hist_v, [lane_off + d], ones, mask=m)

    def reduce_level(width):
        nb = 1 << width

        @plsc.parallel_loop(0, nb // L, unroll=4)
        def _(c):
            acc = hist_v[pl.ds(c * L, L)]
            hist_v[pl.ds(c * L, L)] = zeros
            for l in range(1, L):
                acc = acc + hist_v[pl.ds(l * STRIDE + c * L, L)]
                hist_v[pl.ds(l * STRIDE + c * L, L)] = zeros
            tot_v[pl.ds(c * L, L)] = acc
            s = jnp.sum(acc)
            cvec = jnp.full((L,), c, jnp.int32)
            svec = jnp.full((L,), s, jnp.int32)
            plsc.store_scatter(csum_v, [cvec], svec, mask=lane == 0)

    def find_bucket(width, k_rem):
        nch = (1 << width) // L
        nchv = nch // L

        def body(i, carry):
            cnt, above = carry
            cv = nchv - 1 - i
            v = csum_v[pl.ds(cv * L, L)]
            suf = lax.rev(plsc.cumsum(lax.rev(v, (0,))), (0,)) + above
            cnt = cnt + jnp.sum(jnp.where(suf >= k_rem, 1, 0))
            above = above + jnp.sum(v)
            return cnt, above

        cnt, _ = lax.fori_loop(0, nchv, body, (jnp.int32(0), jnp.int32(0)))
        c0 = cnt - 1

        def body2(cv, acc):
            v = csum_v[pl.ds(cv * L, L)]
            g = cv * L + lane
            return acc + jnp.sum(jnp.where(g > c0, v, 0))

        above_c0 = lax.fori_loop(0, nchv, body2, jnp.int32(0))

        v = tot_v[pl.ds(c0 * L, L)]
        suf = lax.rev(plsc.cumsum(lax.rev(v, (0,))), (0,)) + above_c0
        m = suf >= k_rem
        pc = jnp.sum(jnp.where(m, 1, 0))
        b0 = c0 * L + pc - 1
        s_b0 = jnp.min(jnp.where(m, suf, INT_MAX))
        v_b0 = jnp.sum(jnp.where(lane == pc - 1, v, 0))
        return b0, k_rem - (s_b0 - v_b0)

    lane_cap = lane * CAP

    @plsc.parallel_loop(0, NV, unroll=8, carry=jnp.zeros((L,), jnp.int32))
    def compact(i, cursor):
        v = row_v[pl.ds(i * L, L)]
        m = v >= TG
        bits = lax.bitcast_convert_type(v, jnp.int32)
        plsc.store_scatter(cand_v, [lane_cap + cursor], bits, mask=m)
        return cursor + m.astype(jnp.int32)

    cursor = compact
    n_cand = jnp.sum(cursor)
    max_cand = jnp.max(cursor)
    k0 = jnp.int32(K)

    def cand_select(_):
        trip = jnp.minimum(max_cand, CAP)
        hist_from_cand(24, None, cursor, trip)
        reduce_level(WC)
        b0, k1 = find_bucket(WC, k0)
        hist_from_cand(16, b0, cursor, trip)
        reduce_level(WC)
        b1, k2 = find_bucket(WC, k1)
        p1 = b0 * NBC + b1
        hist_from_cand(8, p1, cursor, trip)
        reduce_level(WC)
        b2, k3 = find_bucket(WC, k2)
        p2 = p1 * NBC + b2
        hist_from_cand(0, p2, cursor, trip)
        reduce_level(WC)
        b3, _ = find_bucket(WC, k3)
        return p2 * NBC + b3

    def full_select(_):
        hist_from_row(W1 + W2, W0, None)
        b0, k1 = find_bucket(W0, k0)
        hist_from_row(W2, W1, b0)
        b1, k2 = find_bucket(W1, k1)
        p1 = b0 * NB1 + b1
        hist_from_row(0, W2, p1)
        b2, _ = find_bucket(W2, k2)
        return p1 * NB2 + b2

    ok = jnp.logical_and(n_cand >= k0, max_cand <= CAP)
    tbits = lax.cond(ok, cand_select, full_select, 0)
    t = lax.bitcast_convert_type(tbits, jnp.float32)

    @plsc.parallel_loop(0, NV, unroll=16)
    def _(i):
        v = row_v[pl.ds(i * L, L)]
        row_v[pl.ds(i * L, L)] = jnp.where(v >= t, v, 0.0)


@functools.partial(
    pl.kernel,
    out_type=jax.ShapeDtypeStruct((B, N), jnp.float32),
    mesh=plsc.VectorSubcoreMesh(core_axis_name="c", subcore_axis_name="s"),
    compiler_params=pltpu.CompilerParams(needs_layout_passes=False),
    scratch_types=[
        pltpu.VMEM((N,), jnp.float32),
        pltpu.VMEM((N,), jnp.float32),
        pltpu.VMEM((HIST_WORDS,), jnp.int32),
        pltpu.VMEM((NB0,), jnp.int32),
        pltpu.VMEM((NB0 // L,), jnp.int32),
        pltpu.VMEM((CAND_WORDS,), jnp.int32),
        pltpu.SemaphoreType.DMA,
        pltpu.SemaphoreType.DMA,
        pltpu.SemaphoreType.DMA,
    ],
)
def _topk_mask_sc(x_hbm, out_hbm, row_a, row_b, hist_v, tot_v, csum_v,
                  cand_v, sem_in0, sem_in1, sem_out):
    wid = lax.axis_index("s") * 2 + lax.axis_index("c")
    lane = lax.iota(jnp.int32, L)
    row0 = wid * ROWS_PER_W
    row1 = row0 + 1

    cp_in0 = pltpu.async_copy(x_hbm.at[row0], row_a, sem_in0)
    cp_in1 = pltpu.async_copy(x_hbm.at[row1], row_b, sem_in1)

    zeros = jnp.zeros((L,), jnp.int32)

    @plsc.parallel_loop(0, HIST_WORDS // L, unroll=8)
    def _(j):
        hist_v[pl.ds(j * L, L)] = zeros

    cp_in0.wait()
    _row_topk(row_a, hist_v, tot_v, csum_v, cand_v, lane)
    cp_out0 = pltpu.async_copy(row_a, out_hbm.at[row0], sem_out)
    cp_in1.wait()
    _row_topk(row_b, hist_v, tot_v, csum_v, cand_v, lane)
    pltpu.sync_copy(row_b, out_hbm.at[row1])
    cp_out0.wait()


def kernel(x):
    return _topk_mask_sc(x)


# speedup vs baseline: 1.2958x; 1.0182x over previous
import functools

import jax
import jax.numpy as jnp
from jax import lax
from jax.experimental import pallas as pl
from jax.experimental.pallas import tpu as pltpu
from jax.experimental.pallas import tpu_sc as plsc

B = 64
N = 32768
K = 2048
L = 16
NV = N // L
NW = 32
ROWS_PER_W = B // NW

W0, W1, W2 = 11, 10, 10
NB0, NB1, NB2 = 1 << W0, 1 << W1, 1 << W2
STRIDE = NB0 + 1
HIST_WORDS = STRIDE * L

TG = 1.25
CAP = 353
CAND_WORDS = L * CAP + NV
WC = 8
NBC = 1 << WC

INT_MAX = 2**31 - 1


def _row_topk(row_v, hist_v, tot_v, csum_v, cand_v, lane):
    lane_off = lane * STRIDE
    ones = jnp.ones((L,), jnp.int32)
    zeros = jnp.zeros((L,), jnp.int32)

    def hist_from_row(shift, width, prefix):
        nb = 1 << width
        base = 0 if prefix is None else prefix * nb

        @plsc.parallel_loop(0, NV, unroll=16)
        def _(i):
            v = row_v[pl.ds(i * L, L)]
            bits = lax.bitcast_convert_type(v, jnp.uint32)
            d = (lax.shift_right_logical(bits, jnp.uint32(shift))
                 - jnp.uint32(base)).astype(jnp.int32)
            m = d.astype(jnp.uint32) < jnp.uint32(nb)
            plsc.addupdate_scatter(hist_v, [lane_off + d], ones, mask=m)

    def hist_from_cand(shift, prefix, cursor, trip):
        base = 0 if prefix is None else prefix * NBC
        lane_cap = lane * CAP

        @plsc.parallel_loop(0, trip, unroll=8)
        def _(j):
            raw = plsc.load_gather(cand_v, [lane_cap + j])
            bits = lax.bitcast_convert_type(raw, jnp.uint32)
            d = (lax.shift_right_logical(bits, jnp.uint32(shift))
                 - jnp.uint32(base)).astype(jnp.int32)
            m = (d.astype(jnp.uint32) < jnp.uint32(NBC)) & (j < cursor)
            plsc.addupdate_scatter(hist_v, [lane_off + d], ones, mask=m)

    def reduce_level(width):
        nb = 1 << width

        @plsc.parallel_loop(0, nb // L, unroll=8)
        def _(c):
            acc = hist_v[pl.ds(c * L, L)]
            hist_v[pl.ds(c * L, L)] = zeros
            for l in range(1, L):
                acc = acc + hist_v[pl.ds(l * STRIDE + c * L, L)]
                hist_v[pl.ds(l * STRIDE + c * L, L)] = zeros
            tot_v[pl.ds(c * L, L)] = acc
            s = jnp.sum(acc)
            cvec = jnp.full((L,), c, jnp.int32)
            svec = jnp.full((L,), s, jnp.int32)
            plsc.store_scatter(csum_v, [cvec], svec, mask=lane == 0)

    def find_bucket(width, k_rem):
        nch = (1 << width) // L
        nchv = nch // L

        def body(i, carry):
            cnt, above = carry
            cv = nchv - 1 - i
            v = csum_v[pl.ds(cv * L, L)]
            suf = lax.rev(plsc.cumsum(lax.rev(v, (0,))), (0,)) + above
            cnt = cnt + jnp.sum(jnp.where(suf >= k_rem, 1, 0))
            above = above + jnp.sum(v)
            return cnt, above

        cnt, _ = lax.fori_loop(0, nchv, body, (jnp.int32(0), jnp.int32(0)))
        c0 = cnt - 1

        def body2(cv, acc):
            v = csum_v[pl.ds(cv * L, L)]
            g = cv * L + lane
            return acc + jnp.sum(jnp.where(g > c0, v, 0))

        above_c0 = lax.fori_loop(0, nchv, body2, jnp.int32(0))

        v = tot_v[pl.ds(c0 * L, L)]
        suf = lax.rev(plsc.cumsum(lax.rev(v, (0,))), (0,)) + above_c0
        m = suf >= k_rem
        pc = jnp.sum(jnp.where(m, 1, 0))
        b0 = c0 * L + pc - 1
        s_b0 = jnp.min(jnp.where(m, suf, INT_MAX))
        v_b0 = jnp.sum(jnp.where(lane == pc - 1, v, 0))
        return b0, k_rem - (s_b0 - v_b0)

    lane_cap = lane * CAP

    @plsc.parallel_loop(0, NV, unroll=16, carry=jnp.zeros((L,), jnp.int32))
    def compact(i, cursor):
        v = row_v[pl.ds(i * L, L)]
        m = v >= TG
        bits = lax.bitcast_convert_type(v, jnp.int32)
        plsc.store_scatter(cand_v, [lane_cap + cursor], bits, mask=m)
        return cursor + m.astype(jnp.int32)

    cursor = compact
    n_cand = jnp.sum(cursor)
    max_cand = jnp.max(cursor)
    k0 = jnp.int32(K)

    def cand_select(_):
        trip = jnp.minimum(max_cand, CAP)
        hist_from_cand(24, None, cursor, trip)
        reduce_level(WC)
        b0, k1 = find_bucket(WC, k0)
        hist_from_cand(16, b0, cursor, trip)
        reduce_level(WC)
        b1, k2 = find_bucket(WC, k1)
        p1 = b0 * NBC + b1
        hist_from_cand(8, p1, cursor, trip)
        reduce_level(WC)
        b2, k3 = find_bucket(WC, k2)
        p2 = p1 * NBC + b2
        hist_from_cand(0, p2, cursor, trip)
        reduce_level(WC)
        b3, _ = find_bucket(WC, k3)
        return p2 * NBC + b3

    def full_select(_):
        hist_from_row(W1 + W2, W0, None)
        b0, k1 = find_bucket(W0, k0)
        hist_from_row(W2, W1, b0)
        b1, k2 = find_bucket(W1, k1)
        p1 = b0 * NB1 + b1
        hist_from_row(0, W2, p1)
        b2, _ = find_bucket(W2, k2)
        return p1 * NB2 + b2

    ok = jnp.logical_and(n_cand >= k0, max_cand <= CAP)
    tbits = lax.cond(ok, cand_select, full_select, 0)
    t = lax.bitcast_convert_type(tbits, jnp.float32)

    @plsc.parallel_loop(0, NV, unroll=16)
    def _(i):
        v = row_v[pl.ds(i * L, L)]
        row_v[pl.ds(i * L, L)] = jnp.where(v >= t, v, 0.0)


@functools.partial(
    pl.kernel,
    out_type=jax.ShapeDtypeStruct((B, N), jnp.float32),
    mesh=plsc.VectorSubcoreMesh(core_axis_name="c", subcore_axis_name="s"),
    compiler_params=pltpu.CompilerParams(needs_layout_passes=False),
    scratch_types=[
        pltpu.VMEM((N,), jnp.float32),
        pltpu.VMEM((N,), jnp.float32),
        pltpu.VMEM((HIST_WORDS,), jnp.int32),
        pltpu.VMEM((NB0,), jnp.int32),
        pltpu.VMEM((NB0 // L,), jnp.int32),
        pltpu.VMEM((CAND_WORDS,), jnp.int32),
        pltpu.SemaphoreType.DMA,
        pltpu.SemaphoreType.DMA,
        pltpu.SemaphoreType.DMA,
    ],
)
def _topk_mask_sc(x_hbm, out_hbm, row_a, row_b, hist_v, tot_v, csum_v,
                  cand_v, sem_in0, sem_in1, sem_out):
    wid = lax.axis_index("s") * 2 + lax.axis_index("c")
    lane = lax.iota(jnp.int32, L)
    row0 = wid * ROWS_PER_W
    row1 = row0 + 1

    cp_in0 = pltpu.async_copy(x_hbm.at[row0], row_a, sem_in0)
    cp_in1 = pltpu.async_copy(x_hbm.at[row1], row_b, sem_in1)

    zeros = jnp.zeros((L,), jnp.int32)

    @plsc.parallel_loop(0, HIST_WORDS // L, unroll=8)
    def _(j):
        hist_v[pl.ds(j * L, L)] = zeros

    cp_in0.wait()
    _row_topk(row_a, hist_v, tot_v, csum_v, cand_v, lane)
    cp_out0 = pltpu.async_copy(row_a, out_hbm.at[row0], sem_out)
    cp_in1.wait()
    _row_topk(row_b, hist_v, tot_v, csum_v, cand_v, lane)
    pltpu.sync_copy(row_b, out_hbm.at[row1])
    cp_out0.wait()


def kernel(x):
    return _topk_mask_sc(x)


# speedup vs baseline: 1.3057x; 1.0077x over previous
import functools

import jax
import jax.numpy as jnp
from jax import lax
from jax.experimental import pallas as pl
from jax.experimental.pallas import tpu as pltpu
from jax.experimental.pallas import tpu_sc as plsc

B = 64
N = 32768
K = 2048
L = 16
NV = N // L
NW = 32
ROWS_PER_W = B // NW

W0, W1, W2 = 11, 10, 10
NB0, NB1, NB2 = 1 << W0, 1 << W1, 1 << W2
STRIDE = NB0 + 1
HIST_WORDS = STRIDE * L

TG = 1.25
CAP = 353
CAND_WORDS = L * CAP + NV
WC = 8
NBC = 1 << WC

INT_MAX = 2**31 - 1


def _row_topk(row_v, hist_v, tot_v, csum_v, cand_v, lane):
    lane_off = lane * STRIDE
    ones = jnp.ones((L,), jnp.int32)
    zeros = jnp.zeros((L,), jnp.int32)

    def hist_from_row(shift, width, prefix):
        nb = 1 << width
        base = 0 if prefix is None else prefix * nb

        @plsc.parallel_loop(0, NV, unroll=16)
        def _(i):
            v = row_v[pl.ds(i * L, L)]
            bits = lax.bitcast_convert_type(v, jnp.uint32)
            d = (lax.shift_right_logical(bits, jnp.uint32(shift))
                 - jnp.uint32(base)).astype(jnp.int32)
            m = d.astype(jnp.uint32) < jnp.uint32(nb)
            plsc.addupdate_scatter(hist_v, [lane_off + d], ones, mask=m)

    def hist_from_cand(shift, prefix, cursor, trip):
        base = 0 if prefix is None else prefix * NBC
        lane_cap = lane * CAP

        @plsc.parallel_loop(0, trip, unroll=8)
        def _(j):
            raw = plsc.load_gather(cand_v, [lane_cap + j])
            bits = lax.bitcast_convert_type(raw, jnp.uint32)
            d = (lax.shift_right_logical(bits, jnp.uint32(shift))
                 - jnp.uint32(base)).astype(jnp.int32)
            m = (d.astype(jnp.uint32) < jnp.uint32(NBC)) & (j < cursor)
            plsc.addupdate_scatter(hist_v, [lane_off + d], ones, mask=m)

    def reduce_level(width):
        nb = 1 << width

        @plsc.parallel_loop(0, nb // L, unroll=8)
        def _(c):
            acc = hist_v[pl.ds(c * L, L)]
            hist_v[pl.ds(c * L, L)] = zeros
            for l in range(1, L):
                acc = acc + hist_v[pl.ds(l * STRIDE + c * L, L)]
                hist_v[pl.ds(l * STRIDE + c * L, L)] = zeros
            tot_v[pl.ds(c * L, L)] = acc
            s = jnp.sum(acc)
            cvec = jnp.full((L,), c, jnp.int32)
            svec = jnp.full((L,), s, jnp.int32)
            plsc.store_scatter(csum_v, [cvec], svec, mask=lane == 0)

    def find_bucket(width, k_rem):
        nch = (1 << width) // L
        nchv = nch // L

        def body(i, carry):
            cnt, above = carry
            cv = nchv - 1 - i
            v = csum_v[pl.ds(cv * L, L)]
            suf = lax.rev(plsc.cumsum(lax.rev(v, (0,))), (0,)) + above
            cnt = cnt + jnp.sum(jnp.where(suf >= k_rem, 1, 0))
            above = above + jnp.sum(v)
            return cnt, above

        cnt, _ = lax.fori_loop(0, nchv, body, (jnp.int32(0), jnp.int32(0)))
        c0 = cnt - 1

        def body2(cv, acc):
            v = csum_v[pl.ds(cv * L, L)]
            g = cv * L + lane
            return acc + jnp.sum(jnp.where(g > c0, v, 0))

        above_c0 = lax.fori_loop(0, nchv, body2, jnp.int32(0))

        v = tot_v[pl.ds(c0 * L, L)]
        suf = lax.rev(plsc.cumsum(lax.rev(v, (0,))), (0,)) + above_c0
        m = suf >= k_rem
        pc = jnp.sum(jnp.where(m, 1, 0))
        b0 = c0 * L + pc - 1
        s_b0 = jnp.min(jnp.where(m, suf, INT_MAX))
        v_b0 = jnp.sum(jnp.where(lane == pc - 1, v, 0))
        return b0, k_rem - (s_b0 - v_b0)

    lane_cap = lane * CAP

    @plsc.parallel_loop(0, NV, unroll=16, carry=jnp.zeros((L,), jnp.int32))
    def compact(i, cursor):
        v = row_v[pl.ds(i * L, L)]
        m = v >= TG
        bits = lax.bitcast_convert_type(v, jnp.int32)
        plsc.store_scatter(cand_v, [lane_cap + cursor], bits, mask=m)
        return cursor + m.astype(jnp.int32)

    cursor = compact
    n_cand = jnp.sum(cursor)
    max_cand = jnp.max(cursor)
    k0 = jnp.int32(K)

    def cand_select(_):
        trip = jnp.minimum(max_cand, CAP)
        hist_from_cand(24, None, cursor, trip)
        reduce_level(WC)
        b0, k1 = find_bucket(WC, k0)
        hist_from_cand(16, b0, cursor, trip)
        reduce_level(WC)
        b1, k2 = find_bucket(WC, k1)
        p1 = b0 * NBC + b1
        hist_from_cand(8, p1, cursor, trip)
        reduce_level(WC)
        b2, k3 = find_bucket(WC, k2)
        p2 = p1 * NBC + b2
        hist_from_cand(0, p2, cursor, trip)
        reduce_level(WC)
        b3, _ = find_bucket(WC, k3)
        return p2 * NBC + b3

    def full_select(_):
        hist_from_row(W1 + W2, W0, None)
        b0, k1 = find_bucket(W0, k0)
        hist_from_row(W2, W1, b0)
        b1, k2 = find_bucket(W1, k1)
        p1 = b0 * NB1 + b1
        hist_from_row(0, W2, p1)
        b2, _ = find_bucket(W2, k2)
        return p1 * NB2 + b2

    ok = jnp.logical_and(n_cand >= k0, max_cand <= CAP)
    tbits = lax.cond(ok, cand_select, full_select, 0)
    return lax.bitcast_convert_type(tbits, jnp.float32)


def _mask_chunk(row_v, t, lo, nv):
    @plsc.parallel_loop(lo, lo + nv, unroll=16)
    def _(i):
        v = row_v[pl.ds(i * L, L)]
        row_v[pl.ds(i * L, L)] = jnp.where(v >= t, v, 0.0)


@functools.partial(
    pl.kernel,
    out_type=jax.ShapeDtypeStruct((B, N), jnp.float32),
    mesh=plsc.VectorSubcoreMesh(core_axis_name="c", subcore_axis_name="s"),
    compiler_params=pltpu.CompilerParams(needs_layout_passes=False),
    scratch_types=[
        pltpu.VMEM((N,), jnp.float32),
        pltpu.VMEM((N,), jnp.float32),
        pltpu.VMEM((HIST_WORDS,), jnp.int32),
        pltpu.VMEM((NB0,), jnp.int32),
        pltpu.VMEM((NB0 // L,), jnp.int32),
        pltpu.VMEM((CAND_WORDS,), jnp.int32),
        pltpu.SemaphoreType.DMA,
        pltpu.SemaphoreType.DMA,
        pltpu.SemaphoreType.DMA,
    ],
)
def _topk_mask_sc(x_hbm, out_hbm, row_a, row_b, hist_v, tot_v, csum_v,
                  cand_v, sem_in0, sem_in1, sem_out):
    wid = lax.axis_index("s") * 2 + lax.axis_index("c")
    lane = lax.iota(jnp.int32, L)
    row0 = wid * ROWS_PER_W
    row1 = row0 + 1

    cp_in0 = pltpu.async_copy(x_hbm.at[row0], row_a, sem_in0)
    cp_in1 = pltpu.async_copy(x_hbm.at[row1], row_b, sem_in1)

    zeros = jnp.zeros((L,), jnp.int32)

    @plsc.parallel_loop(0, HIST_WORDS // L, unroll=8)
    def _(j):
        hist_v[pl.ds(j * L, L)] = zeros

    cp_in0.wait()
    t0 = _row_topk(row_a, hist_v, tot_v, csum_v, cand_v, lane)
    _mask_chunk(row_a, t0, 0, NV)
    cp_out0 = pltpu.async_copy(row_a, out_hbm.at[row0], sem_out)
    cp_in1.wait()
    t1 = _row_topk(row_b, hist_v, tot_v, csum_v, cand_v, lane)
    nvc = NV // 4
    cps = []
    for q in range(4):
        _mask_chunk(row_b, t1, q * nvc, nvc)
        cps.append(pltpu.async_copy(
            row_b.at[pl.ds(q * nvc * L, nvc * L)],
            out_hbm.at[row1, pl.ds(q * nvc * L, nvc * L)],
            sem_in0 if q % 2 == 0 else sem_in1))
    for cp in cps:
        cp.wait()
    cp_out0.wait()


def kernel(x):
    return _topk_mask_sc(x)
